# Initial kernel scaffold; baseline (speedup 1.0000x reference)
import functools

import jax
import jax.numpy as jnp
from jax import lax
from jax.experimental import pallas as pl
from jax.experimental.pallas import tpu as pltpu
from jax.experimental.pallas import tpu_sc as plsc


_NC = 2
_NS = 16
_NW = _NC * _NS


def _make_edge_agg(n, d, c_chunks, k):
    rt = n // _NS
    mesh = plsc.VectorSubcoreMesh(core_axis_name="c", subcore_axis_name="s")

    def body(table, srcw, dstw, zeros, out, src_v, dst_v, rows_v, acc, sem):
        cid = lax.axis_index("c")
        sid = lax.axis_index("s")
        pltpu.sync_copy(zeros, acc.at[pl.ds(sid * rt, rt)])
        pltpu.sync_copy(srcw.at[cid, sid], src_v)
        pltpu.sync_copy(dstw.at[cid, sid], dst_v)
        plsc.subcore_barrier()

        def step(j, carry):
            pltpu.async_copy(table.at[src_v.at[j]], rows_v, sem).wait()
            pltpu.sync_copy(rows_v, acc.at[dst_v.at[j]], add=True)
            return carry

        lax.fori_loop(0, c_chunks, step, 0)
        plsc.subcore_barrier()
        pltpu.sync_copy(acc.at[pl.ds(sid * rt, rt)],
                        out.at[cid, pl.ds(sid * rt, rt)])

    return pl.kernel(
        body,
        mesh=mesh,
        out_type=jax.ShapeDtypeStruct((_NC, n, d), jnp.float32),
        scratch_types=[
            pltpu.VMEM((c_chunks, k), jnp.int32),
            pltpu.VMEM((c_chunks, k), jnp.int32),
            pltpu.VMEM((k, d), jnp.float32),
            pltpu.VMEM_SHARED((n, d), jnp.float32),
            pltpu.SemaphoreType.DMA,
        ],
    )


def _make_degree(n, d, c_chunks, k):
    rt = n // _NS
    mesh = plsc.VectorSubcoreMesh(core_axis_name="c", subcore_axis_name="s")

    def body(dstw, zeros, ones, out, dst_v, ones_v, acc):
        cid = lax.axis_index("c")
        sid = lax.axis_index("s")
        pltpu.sync_copy(zeros, acc.at[pl.ds(sid * rt, rt)])
        pltpu.sync_copy(ones, ones_v)
        pltpu.sync_copy(dstw.at[cid, sid], dst_v)
        plsc.subcore_barrier()

        def step(j, carry):
            pltpu.sync_copy(ones_v, acc.at[dst_v.at[j]], add=True)
            return carry

        lax.fori_loop(0, c_chunks, step, 0)
        plsc.subcore_barrier()
        pltpu.sync_copy(acc.at[pl.ds(sid * rt, rt)],
                        out.at[cid, pl.ds(sid * rt, rt)])

    return pl.kernel(
        body,
        mesh=mesh,
        out_type=jax.ShapeDtypeStruct((_NC, n, d), jnp.float32),
        scratch_types=[
            pltpu.VMEM((c_chunks, k), jnp.int32),
            pltpu.VMEM((k, d), jnp.float32),
            pltpu.VMEM_SHARED((n, d), jnp.float32),
        ],
    )


def _tc1_body(x_ref, w1_ref, degp_ref, g1_ref, dinv_ref):
    deg = 1.0 + degp_ref[0] + degp_ref[1]
    dinv = lax.rsqrt(deg)
    h = jnp.dot(x_ref[...], w1_ref[...], preferred_element_type=jnp.float32)
    g1_ref[...] = dinv * h
    dinv_ref[...] = dinv


def _tc2_body(dinv_ref, p_ref, g1_ref, b1_ref, w2_ref, g2_ref):
    agg = p_ref[0] + p_ref[1] + g1_ref[...]
    out1 = jnp.maximum(dinv_ref[...] * agg + b1_ref[...], 0.0)
    h2 = jnp.dot(out1, w2_ref[...], preferred_element_type=jnp.float32)
    g2_ref[...] = dinv_ref[...] * h2


def _tc3_body(dinv_ref, p_ref, g2_ref, b2_ref, o_ref):
    out2 = dinv_ref[...] * (p_ref[0] + p_ref[1] + g2_ref[...]) + b2_ref[...]
    m = jnp.max(out2, axis=1, keepdims=True)
    lse = m + jnp.log(jnp.sum(jnp.exp(out2 - m), axis=1, keepdims=True))
    o_ref[...] = out2 - lse


@jax.jit
def kernel(x, edge_index, dropout, W1, b1, W2, b2):
    n, f = x.shape
    e = edge_index.shape[1]
    dh = W1.shape[1]
    nc = W2.shape[1]

    k = 125
    per_w = e // _NW
    c_chunks = per_w // k
    srcw = edge_index[0].reshape(_NC, _NS, c_chunks, k)
    dstw = edge_index[1].reshape(_NC, _NS, c_chunks, k)

    rt = n // _NS
    zeros1 = jnp.zeros((rt, 1), jnp.float32)
    ones_k = jnp.ones((k, 1), jnp.float32)
    zeros_h = jnp.zeros((rt, dh), jnp.float32)
    zeros_c = jnp.zeros((rt, nc), jnp.float32)

    degp = _make_degree(n, 1, c_chunks, k)(dstw, zeros1, ones_k)

    r = 2000
    grid = n // r
    g1, dinv = pl.pallas_call(
        _tc1_body,
        grid=(grid,),
        in_specs=[
            pl.BlockSpec((r, f), lambda i: (i, 0)),
            pl.BlockSpec((f, dh), lambda i: (0, 0)),
            pl.BlockSpec((_NC, r, 1), lambda i: (0, i, 0)),
        ],
        out_specs=[
            pl.BlockSpec((r, dh), lambda i: (i, 0)),
            pl.BlockSpec((r, 1), lambda i: (i, 0)),
        ],
        out_shape=[
            jax.ShapeDtypeStruct((n, dh), jnp.float32),
            jax.ShapeDtypeStruct((n, 1), jnp.float32),
        ],
    )(x, W1, degp)

    p1 = _make_edge_agg(n, dh, c_chunks, k)(g1, srcw, dstw, zeros_h)

    g2 = pl.pallas_call(
        _tc2_body,
        grid=(grid,),
        in_specs=[
            pl.BlockSpec((r, 1), lambda i: (i, 0)),
            pl.BlockSpec((_NC, r, dh), lambda i: (0, i, 0)),
            pl.BlockSpec((r, dh), lambda i: (i, 0)),
            pl.BlockSpec((1, dh), lambda i: (0, 0)),
            pl.BlockSpec((dh, nc), lambda i: (0, 0)),
        ],
        out_specs=pl.BlockSpec((r, nc), lambda i: (i, 0)),
        out_shape=jax.ShapeDtypeStruct((n, nc), jnp.float32),
    )(dinv, p1, g1, b1.reshape(1, dh), W2)

    p2 = _make_edge_agg(n, nc, c_chunks, k)(g2, srcw, dstw, zeros_c)

    out = pl.pallas_call(
        _tc3_body,
        grid=(grid,),
        in_specs=[
            pl.BlockSpec((r, 1), lambda i: (i, 0)),
            pl.BlockSpec((_NC, r, nc), lambda i: (0, i, 0)),
            pl.BlockSpec((r, nc), lambda i: (i, 0)),
            pl.BlockSpec((1, nc), lambda i: (0, 0)),
        ],
        out_specs=pl.BlockSpec((r, nc), lambda i: (i, 0)),
        out_shape=jax.ShapeDtypeStruct((n, nc), jnp.float32),
    )(dinv, p2, g2, b2.reshape(1, nc))

    return out


# speedup vs baseline: 31.4498x; 31.4498x over previous
import functools

import jax
import jax.numpy as jnp
from jax import lax
from jax.experimental import pallas as pl
from jax.experimental.pallas import tpu as pltpu
from jax.experimental.pallas import tpu_sc as plsc


_NC = 2
_NS = 16
_NW = _NC * _NS


def _make_edge_agg(n_pad, d, c_chunks, k):
    rt = n_pad // _NS
    mesh = plsc.VectorSubcoreMesh(core_axis_name="c", subcore_axis_name="s")

    g_stage = 40
    n_groups = c_chunks // g_stage

    def body(table, srcw, dstw, zeros, out, src_v, dst_v, rows_v, acc, sem):
        cid = lax.axis_index("c")
        sid = lax.axis_index("s")
        pltpu.sync_copy(zeros, acc.at[pl.ds(sid * rt, rt)])
        plsc.subcore_barrier()

        def outer(g, carry):
            pltpu.sync_copy(srcw.at[cid, sid, pl.ds(g * g_stage, g_stage)],
                            src_v)
            pltpu.sync_copy(dstw.at[cid, sid, pl.ds(g * g_stage, g_stage)],
                            dst_v)

            def step(j, carry2):
                pltpu.async_copy(table.at[src_v.at[j]], rows_v, sem).wait()
                pltpu.sync_copy(rows_v, acc.at[dst_v.at[j]], add=True)
                return carry2

            lax.fori_loop(0, g_stage, step, 0)
            return carry

        lax.fori_loop(0, n_groups, outer, 0)
        plsc.subcore_barrier()
        pltpu.sync_copy(acc.at[pl.ds(sid * rt, rt)],
                        out.at[cid, pl.ds(sid * rt, rt)])

    return pl.kernel(
        body,
        mesh=mesh,
        out_type=jax.ShapeDtypeStruct((_NC, n_pad, d), jnp.float32),
        compiler_params=pltpu.CompilerParams(use_tc_tiling_on_sc=False),
        scratch_types=[
            pltpu.VMEM((g_stage, k), jnp.int32),
            pltpu.VMEM((g_stage, k), jnp.int32),
            pltpu.VMEM((k, d), jnp.float32),
            pltpu.VMEM_SHARED((n_pad, d), jnp.float32),
            pltpu.SemaphoreType.DMA,
        ],
    )


def _make_degree(n_pad, d, c_chunks, k):
    rt = n_pad // _NS
    mesh = plsc.VectorSubcoreMesh(core_axis_name="c", subcore_axis_name="s")

    def body(dstw, zeros, ones, out, dst_v, ones_v, acc):
        cid = lax.axis_index("c")
        sid = lax.axis_index("s")
        pltpu.sync_copy(zeros, acc.at[pl.ds(sid * rt, rt)])
        pltpu.sync_copy(ones, ones_v)
        pltpu.sync_copy(dstw.at[cid, sid], dst_v)
        plsc.subcore_barrier()

        def step(j, carry):
            pltpu.sync_copy(ones_v, acc.at[dst_v.at[j]], add=True)
            return carry

        lax.fori_loop(0, c_chunks, step, 0)
        plsc.subcore_barrier()
        pltpu.sync_copy(acc.at[pl.ds(sid * rt, rt)],
                        out.at[cid, pl.ds(sid * rt, rt)])

    return pl.kernel(
        body,
        mesh=mesh,
        out_type=jax.ShapeDtypeStruct((_NC, n_pad, d), jnp.float32),
        compiler_params=pltpu.CompilerParams(use_tc_tiling_on_sc=False),
        scratch_types=[
            pltpu.VMEM((c_chunks, k), jnp.int32),
            pltpu.VMEM((k, d), jnp.float32),
            pltpu.VMEM_SHARED((n_pad, d), jnp.float32),
        ],
    )


def _tc1_body(x_ref, w1_ref, degp_ref, g1_ref, dinv_ref):
    deg = 1.0 + degp_ref[0][:, 0:1] + degp_ref[1][:, 0:1]
    dinv = lax.rsqrt(deg)
    h = jnp.dot(x_ref[...], w1_ref[...], preferred_element_type=jnp.float32)
    g1_ref[...] = dinv * h
    dinv_ref[...] = dinv


def _tc2_body(dinv_ref, p_ref, g1_ref, b1_ref, w2_ref, g2_ref):
    agg = p_ref[0] + p_ref[1] + g1_ref[...]
    out1 = jnp.maximum(dinv_ref[...] * agg + b1_ref[...], 0.0)
    h2 = jnp.dot(out1, w2_ref[...], preferred_element_type=jnp.float32)
    g2_ref[...] = dinv_ref[...] * h2


def _tc3_body(dinv_ref, p_ref, g2_ref, b2_ref, o_ref):
    full = dinv_ref[...] * (p_ref[0] + p_ref[1] + g2_ref[...]) + b2_ref[...]
    out2 = full[:, 0:2]
    m = jnp.max(out2, axis=1, keepdims=True)
    lse = m + jnp.log(jnp.sum(jnp.exp(out2 - m), axis=1, keepdims=True))
    o_ref[...] = out2 - lse


@jax.jit
def kernel(x, edge_index, dropout, W1, b1, W2, b2):
    n, f = x.shape
    e = edge_index.shape[1]
    dh = W1.shape[1]
    nc = W2.shape[1]

    k = 125
    per_w = e // _NW
    c_chunks = per_w // k
    srcw = edge_index[0].reshape(_NC, _NS, c_chunks, k)
    dstw = edge_index[1].reshape(_NC, _NS, c_chunks, k)

    n_pad = ((n + 127) // 128) * 128
    rt = n_pad // _NS
    dp = 8
    zeros8 = jnp.zeros((rt, dp), jnp.float32)
    ones_k = jnp.ones((k, dp), jnp.float32)
    zeros_h = jnp.zeros((rt, dh), jnp.float32)
    W2p = jnp.pad(W2, ((0, 0), (0, dp - nc)))
    b2p = jnp.pad(b2, (0, dp - nc))

    degp = _make_degree(n_pad, dp, c_chunks, k)(dstw, zeros8, ones_k)

    r = 2000
    grid = n // r
    g1, dinv = pl.pallas_call(
        _tc1_body,
        grid=(grid,),
        in_specs=[
            pl.BlockSpec((r, f), lambda i: (i, 0)),
            pl.BlockSpec((f, dh), lambda i: (0, 0)),
            pl.BlockSpec((_NC, r, 8), lambda i: (0, i, 0)),
        ],
        out_specs=[
            pl.BlockSpec((r, dh), lambda i: (i, 0)),
            pl.BlockSpec((r, 1), lambda i: (i, 0)),
        ],
        out_shape=[
            jax.ShapeDtypeStruct((n, dh), jnp.float32),
            jax.ShapeDtypeStruct((n, 1), jnp.float32),
        ],
    )(x, W1, degp)

    p1 = _make_edge_agg(n_pad, dh, c_chunks, k)(g1, srcw, dstw, zeros_h)

    g2 = pl.pallas_call(
        _tc2_body,
        grid=(grid,),
        in_specs=[
            pl.BlockSpec((r, 1), lambda i: (i, 0)),
            pl.BlockSpec((_NC, r, dh), lambda i: (0, i, 0)),
            pl.BlockSpec((r, dh), lambda i: (i, 0)),
            pl.BlockSpec((1, dh), lambda i: (0, 0)),
            pl.BlockSpec((dh, dp), lambda i: (0, 0)),
        ],
        out_specs=pl.BlockSpec((r, dp), lambda i: (i, 0)),
        out_shape=jax.ShapeDtypeStruct((n, dp), jnp.float32),
    )(dinv, p1, g1, b1.reshape(1, dh), W2p)

    p2 = _make_edge_agg(n_pad, dp, c_chunks, k)(g2, srcw, dstw, zeros8)

    out = pl.pallas_call(
        _tc3_body,
        grid=(grid,),
        in_specs=[
            pl.BlockSpec((r, 1), lambda i: (i, 0)),
            pl.BlockSpec((_NC, r, dp), lambda i: (0, i, 0)),
            pl.BlockSpec((r, dp), lambda i: (i, 0)),
            pl.BlockSpec((1, dp), lambda i: (0, 0)),
        ],
        out_specs=pl.BlockSpec((r, nc), lambda i: (i, 0)),
        out_shape=jax.ShapeDtypeStruct((n, nc), jnp.float32),
    )(dinv, p2, g2, b2p.reshape(1, dp))

    return out


# speedup vs baseline: 45.8892x; 1.4591x over previous
import jax
import jax.numpy as jnp
from jax import lax
from jax.experimental import pallas as pl
from jax.experimental.pallas import tpu as pltpu
from jax.experimental.pallas import tpu_sc as plsc


_NC = 2
_NS = 16
_NW = _NC * _NS
_K = 128
_GRP = 28


def _make_edge_agg(n_pad, d, gpw):
    rt = n_pad // _NS
    spw = gpw // 2
    ngrp = gpw // _GRP
    sgrp = _GRP // 2
    mesh = plsc.VectorSubcoreMesh(core_axis_name="c", subcore_axis_name="s")

    def body(table, srcw, dstw2, zeros, out,
             src_v, dst_v, rows0, rows1, acc, s0, s1, s2, s3):
        cid = lax.axis_index("c")
        sid = lax.axis_index("s")
        w = cid * _NS + sid
        pltpu.sync_copy(zeros, acc.at[pl.ds(sid * rt, rt)])
        plsc.subcore_barrier()

        def group(g, carry):
            pltpu.sync_copy(srcw.at[pl.ds(w * gpw + g * _GRP, _GRP)], src_v)
            pltpu.sync_copy(dstw2.at[pl.ds(w * spw + g * sgrp, sgrp)], dst_v)

            def pair(u2, c2):
                u = 2 * u2
                ga = pltpu.async_copy(table.at[src_v.at[2 * u]],
                                      rows0.at[pl.ds(0, _K)], s0)
                gb = pltpu.async_copy(table.at[src_v.at[2 * u + 1]],
                                      rows0.at[pl.ds(_K, _K)], s1)
                gc = pltpu.async_copy(table.at[src_v.at[2 * u + 2]],
                                      rows1.at[pl.ds(0, _K)], s2)
                gd = pltpu.async_copy(table.at[src_v.at[2 * u + 3]],
                                      rows1.at[pl.ds(_K, _K)], s3)
                ga.wait()
                gb.wait()
                pltpu.sync_copy(rows0, acc.at[dst_v.at[u]], add=True)
                gc.wait()
                gd.wait()
                pltpu.sync_copy(rows1, acc.at[dst_v.at[u + 1]], add=True)
                return c2

            lax.fori_loop(0, sgrp // 2, pair, 0)
            return carry

        lax.fori_loop(0, ngrp, group, 0)
        plsc.subcore_barrier()
        pltpu.sync_copy(acc.at[pl.ds(sid * rt, rt)],
                        out.at[cid, pl.ds(sid * rt, rt)])

    return pl.kernel(
        body,
        mesh=mesh,
        out_type=jax.ShapeDtypeStruct((_NC, n_pad, d), jnp.float32),
        compiler_params=pltpu.CompilerParams(use_tc_tiling_on_sc=False),
        scratch_types=[
            pltpu.VMEM((_GRP, _K), jnp.int32),
            pltpu.VMEM((sgrp, 2 * _K), jnp.int32),
            pltpu.VMEM((2 * _K, d), jnp.float32),
            pltpu.VMEM((2 * _K, d), jnp.float32),
            pltpu.VMEM_SHARED((n_pad, d), jnp.float32),
            pltpu.SemaphoreType.DMA,
            pltpu.SemaphoreType.DMA,
            pltpu.SemaphoreType.DMA,
            pltpu.SemaphoreType.DMA,
        ],
    )


def _make_degree(n_pad, d, gpw):
    rt = n_pad // _NS
    spw = gpw // 2
    ngrp = gpw // _GRP
    sgrp = _GRP // 2
    mesh = plsc.VectorSubcoreMesh(core_axis_name="c", subcore_axis_name="s")

    def body(dstw2, zeros, ones, out, dst_v, ones_v, acc):
        cid = lax.axis_index("c")
        sid = lax.axis_index("s")
        w = cid * _NS + sid
        pltpu.sync_copy(zeros, acc.at[pl.ds(sid * rt, rt)])
        pltpu.sync_copy(ones, ones_v)
        plsc.subcore_barrier()

        def group(g, carry):
            pltpu.sync_copy(dstw2.at[pl.ds(w * spw + g * sgrp, sgrp)], dst_v)

            def step(u, c2):
                pltpu.sync_copy(ones_v, acc.at[dst_v.at[u]], add=True)
                return c2

            lax.fori_loop(0, sgrp, step, 0)
            return carry

        lax.fori_loop(0, ngrp, group, 0)
        plsc.subcore_barrier()
        pltpu.sync_copy(acc.at[pl.ds(sid * rt, rt)],
                        out.at[cid, pl.ds(sid * rt, rt)])

    return pl.kernel(
        body,
        mesh=mesh,
        out_type=jax.ShapeDtypeStruct((_NC, n_pad, d), jnp.float32),
        compiler_params=pltpu.CompilerParams(use_tc_tiling_on_sc=False),
        scratch_types=[
            pltpu.VMEM((sgrp, 2 * _K), jnp.int32),
            pltpu.VMEM((2 * _K, d), jnp.float32),
            pltpu.VMEM_SHARED((n_pad, d), jnp.float32),
        ],
    )


def _tc1_body(x4_ref, w1blk_ref, dv4_ref, g1_ref):
    dinv4 = lax.rsqrt(1.0 + dv4_ref[0] + dv4_ref[1])
    h4 = jnp.dot(x4_ref[...], w1blk_ref[...],
                 preferred_element_type=jnp.float32)
    g1_ref[...] = dinv4 * h4


def _tc2_body(dv4_ref, p_ref, g1_ref, b1t4_ref, w2big_ref, g2_ref):
    dinv4 = lax.rsqrt(1.0 + dv4_ref[0] + dv4_ref[1])
    agg = p_ref[0] + p_ref[1] + g1_ref[...]
    out1 = jnp.maximum(dinv4 * agg + b1t4_ref[...], 0.0)
    g2_ref[...] = jnp.dot(dinv4 * out1, w2big_ref[...],
                          preferred_element_type=jnp.float32)


def _tc3_body(dv4_ref, p_ref, g2_ref, b2t4_ref, o_ref):
    r4 = g2_ref.shape[0]
    dinv4 = lax.rsqrt(1.0 + dv4_ref[0] + dv4_ref[1])
    z = dinv4 * (p_ref[0] + p_ref[1] + g2_ref[...]) + b2t4_ref[...]
    lane = lax.broadcasted_iota(jnp.int32, (r4, 128), 1)
    other = jnp.where(lane % 32 == 0, jnp.roll(z, -1, axis=1),
                      jnp.roll(z, 1, axis=1))
    m = jnp.maximum(z, other)
    o_ref[...] = z - m - jnp.log(jnp.exp(z - m) + jnp.exp(other - m))


@jax.jit
def kernel(x, edge_index, dropout, W1, b1, W2, b2):
    n, f = x.shape
    e = edge_index.shape[1]
    dh = W1.shape[1]
    nc = W2.shape[1]

    n_pad = ((n + 127) // 128) * 128
    rt = n_pad // _NS

    chunks = (e + _K - 1) // _K
    gpw = (chunks + _NW - 1) // _NW
    gpw = ((gpw + _GRP - 1) // _GRP) * _GRP
    e_pad = _NW * gpw * _K
    src = jnp.concatenate(
        [edge_index[0], jnp.zeros((e_pad - e,), jnp.int32)])
    dst = jnp.concatenate(
        [edge_index[1], jnp.full((e_pad - e,), n_pad - 8, jnp.int32)])
    srcw = src.reshape(_NW * gpw, _K)
    dstw2 = dst.reshape(_NW * gpw // 2, 2 * _K)

    zeros_h = jnp.zeros((rt, dh), jnp.float32)
    ones_v = jnp.ones((2 * _K, dh), jnp.float32)

    eye4 = jnp.eye(4, dtype=jnp.float32)
    w1blk = jnp.kron(eye4, W1)
    w2big = jnp.kron(eye4, jnp.pad(W2, ((0, 0), (0, dh - nc))))
    b1t4 = jnp.tile(b1, 4).reshape(1, 128)
    b2t4 = jnp.tile(jnp.pad(b2, (0, dh - nc)), 4).reshape(1, 128)

    x4 = jnp.pad(x, ((0, n_pad - n), (0, 0))).reshape(n_pad // 4, 4 * f)

    r = 2176
    grid = n_pad // r
    r4 = r // 4

    degp = _make_degree(n_pad, dh, gpw)(dstw2, zeros_h, ones_v)
    dv4 = degp.reshape(_NC, n_pad // 4, 128)

    g1p4 = pl.pallas_call(
        _tc1_body,
        grid=(grid,),
        in_specs=[
            pl.BlockSpec((r4, 4 * f), lambda i: (i, 0)),
            pl.BlockSpec((4 * f, 128), lambda i: (0, 0)),
            pl.BlockSpec((_NC, r4, 128), lambda i: (0, i, 0)),
        ],
        out_specs=pl.BlockSpec((r4, 128), lambda i: (i, 0)),
        out_shape=jax.ShapeDtypeStruct((n_pad // 4, 128), jnp.float32),
    )(x4, w1blk, dv4)

    p1 = _make_edge_agg(n_pad, dh, gpw)(
        g1p4.reshape(n_pad, dh), srcw, dstw2, zeros_h)
    p1v = p1.reshape(_NC, n_pad // 4, 128)

    g2big = pl.pallas_call(
        _tc2_body,
        grid=(grid,),
        in_specs=[
            pl.BlockSpec((_NC, r4, 128), lambda i: (0, i, 0)),
            pl.BlockSpec((_NC, r4, 128), lambda i: (0, i, 0)),
            pl.BlockSpec((r4, 128), lambda i: (i, 0)),
            pl.BlockSpec((1, 128), lambda i: (0, 0)),
            pl.BlockSpec((128, 128), lambda i: (0, 0)),
        ],
        out_specs=pl.BlockSpec((r4, 128), lambda i: (i, 0)),
        out_shape=jax.ShapeDtypeStruct((n_pad // 4, 128), jnp.float32),
    )(dv4, p1v, g1p4, b1t4, w2big)

    p2 = _make_edge_agg(n_pad, dh, gpw)(
        g2big.reshape(n_pad, dh), srcw, dstw2, zeros_h)
    p2v = p2.reshape(_NC, n_pad // 4, 128)

    zfull = pl.pallas_call(
        _tc3_body,
        grid=(grid,),
        in_specs=[
            pl.BlockSpec((_NC, r4, 128), lambda i: (0, i, 0)),
            pl.BlockSpec((_NC, r4, 128), lambda i: (0, i, 0)),
            pl.BlockSpec((r4, 128), lambda i: (i, 0)),
            pl.BlockSpec((1, 128), lambda i: (0, 0)),
        ],
        out_specs=pl.BlockSpec((r4, 128), lambda i: (i, 0)),
        out_shape=jax.ShapeDtypeStruct((n_pad // 4, 128), jnp.float32),
    )(dv4, p2v, g2big, b2t4)

    return zfull.reshape(n_pad, dh)[:n, :nc]
